# baseline (speedup 1.0000x reference)
import jax
import jax.numpy as jnp
from jax import lax
from jax.experimental import pallas as pl
from jax.experimental.pallas import tpu as pltpu
from jax.experimental.pallas import tpu_sc as plsc

D_MODEL = 1024
MAX_SEQ_LEN = 2048
BATCH = 16
NC = 2
NS = 16
NW = NC * NS
LANES = 16

SPAN = BATCH * MAX_SEQ_LEN // NW
PARTS = MAX_SEQ_LEN // SPAN
CHUNK = 64
STEPS = SPAN // CHUNK


def _sc_body(len_hbm, table_hbm, emb_hbm, pos_hbm, len_v, idx_v, rows_v, sem):
    wid = lax.axis_index("s") * NC + lax.axis_index("c")
    b = wid // PARTS
    part = wid % PARTS
    pltpu.sync_copy(len_hbm, len_v)
    lanes = lax.iota(jnp.int32, LANES)
    bvec = jnp.full((LANES,), b, dtype=jnp.int32)
    len_b = len_v[...].at[bvec].get(mode="promise_in_bounds")

    def step_fn(s, carry):
        base = part * SPAN + s * CHUNK
        for k in range(CHUNK // LANES):
            col = base + k * LANES + lanes
            idx_v[pl.ds(k * LANES, LANES)] = jnp.where(col < len_b, col + 1, 0)
        pltpu.async_copy(table_hbm.at[idx_v], rows_v, sem).wait()
        pltpu.sync_copy(rows_v, emb_hbm.at[b, pl.ds(base, CHUNK)])
        pltpu.sync_copy(idx_v, pos_hbm.at[b, pl.ds(base, CHUNK)])
        return carry

    lax.fori_loop(0, STEPS, step_fn, 0)


def kernel(input_len, pe_table):
    mesh = plsc.VectorSubcoreMesh(core_axis_name="c", subcore_axis_name="s")
    f = pl.kernel(
        _sc_body,
        out_type=[
            jax.ShapeDtypeStruct((BATCH, MAX_SEQ_LEN, D_MODEL), jnp.float32),
            jax.ShapeDtypeStruct((BATCH, MAX_SEQ_LEN), jnp.int32),
        ],
        mesh=mesh,
        scratch_types=[
            pltpu.VMEM((LANES,), jnp.int32),
            pltpu.VMEM((CHUNK,), jnp.int32),
            pltpu.VMEM((CHUNK, D_MODEL), jnp.float32),
            pltpu.SemaphoreType.DMA,
        ],
    )
    emb, pos = f(input_len, pe_table)
    return emb, pos
